# baseline (speedup 1.0000x reference)
import jax
import jax.numpy as jnp
from jax.experimental import pallas as pl
from jax.experimental.pallas import tpu as pltpu

_N_SAMPLES = 8
_BATCH = 4096
_C = 1000
_BB = 256
_NB = _BATCH // _BB


def _sampler_kernel(dist_ref, gu_ref, out_ref):
    d = dist_ref[...]
    s = jnp.sum(d, axis=0, keepdims=True)
    p = d / s
    logp = jnp.log(p + 1e-9)
    ent = -jnp.sum(p * logp, axis=0, keepdims=True)

    sub = jax.lax.broadcasted_iota(jnp.int32, (8, _BB), 0)
    for n in range(_N_SAMPLES):
        def _tile_k(t):
            u = gu_ref[n, 8 * t:8 * t + 8, :]
            lp = logp[8 * t:8 * t + 8, :]
            return lp + -jnp.log(-jnp.log(u)), lp

        def _chain(t0, t1):
            k_acc, lp_acc = _tile_k(t0)
            t_acc = jnp.full((8, _BB), t0, jnp.int32)
            for t in range(t0 + 1, t1):
                kt, lpt = _tile_k(t)
                better = kt > k_acc
                k_acc = jnp.maximum(kt, k_acc)
                lp_acc = jnp.where(better, lpt, lp_acc)
                t_acc = jnp.where(better, jnp.int32(t), t_acc)
            return k_acc, lp_acc, t_acc

        nt = _C // 8
        bounds = [0, nt // 4, nt // 2, 3 * nt // 4, nt]
        chains = [_chain(bounds[j], bounds[j + 1]) for j in range(4)]

        def _merge(a, b):
            ka, lpa, ta = a
            kb, lpb, tb = b
            bet = kb > ka
            return (jnp.maximum(kb, ka), jnp.where(bet, lpb, lpa),
                    jnp.where(bet, tb, ta))

        k_acc, lp_acc, t_acc = _merge(_merge(chains[0], chains[1]),
                                      _merge(chains[2], chains[3]))

        m = jnp.max(k_acc, axis=0, keepdims=True)
        c_acc = t_acc * 8 + sub
        cbest = jnp.min(jnp.where(k_acc == m, c_acc, jnp.int32(1 << 30)),
                        axis=0, keepdims=True)
        log_prob = jnp.sum(jnp.where(c_acc == cbest, lp_acc, 0.0),
                           axis=0, keepdims=True)
        out_ref[n, :] = (log_prob + ent)[0, :]


@jax.jit
def kernel(dist, gumbel_u):
    dist_t = jnp.transpose(dist)
    gu_t = jnp.transpose(gumbel_u, (0, 2, 1))
    return pl.pallas_call(
        _sampler_kernel,
        grid=(_NB,),
        in_specs=[
            pl.BlockSpec((_C, _BB), lambda i: (0, i)),
            pl.BlockSpec((_N_SAMPLES, _C, _BB), lambda i: (0, 0, i)),
        ],
        out_specs=pl.BlockSpec((_N_SAMPLES, _BB), lambda i: (0, i)),
        out_shape=jax.ShapeDtypeStruct((_N_SAMPLES, _BATCH), jnp.float32),
    )(dist_t, gu_t)


# speedup vs baseline: 1.0099x; 1.0099x over previous
import jax
import jax.numpy as jnp
from jax.experimental import pallas as pl
from jax.experimental.pallas import tpu as pltpu

_N_SAMPLES = 8
_BATCH = 4096
_C = 1000
_BB = 512
_NB = _BATCH // _BB


def _sampler_kernel(dist_ref, gu_ref, out_ref):
    d = dist_ref[...]
    s = jnp.sum(d, axis=0, keepdims=True)
    p = d / s
    logp = jnp.log(p + 1e-9)
    ent = -jnp.sum(p * logp, axis=0, keepdims=True)

    sub = jax.lax.broadcasted_iota(jnp.int32, (8, _BB), 0)
    for n in range(_N_SAMPLES):
        def _tile_k(t):
            u = gu_ref[n, 8 * t:8 * t + 8, :]
            lp = logp[8 * t:8 * t + 8, :]
            return lp + -jnp.log(-jnp.log(u)), lp

        def _chain(t0, t1):
            k_acc, lp_acc = _tile_k(t0)
            t_acc = jnp.full((8, _BB), t0, jnp.int32)
            for t in range(t0 + 1, t1):
                kt, lpt = _tile_k(t)
                better = kt > k_acc
                k_acc = jnp.maximum(kt, k_acc)
                lp_acc = jnp.where(better, lpt, lp_acc)
                t_acc = jnp.where(better, jnp.int32(t), t_acc)
            return k_acc, lp_acc, t_acc

        half = (_C // 8) // 2
        ka, lpa, ta = _chain(0, half)
        kb, lpb, tb = _chain(half, _C // 8)
        bet = kb > ka
        k_acc = jnp.maximum(kb, ka)
        lp_acc = jnp.where(bet, lpb, lpa)
        t_acc = jnp.where(bet, tb, ta)

        m = jnp.max(k_acc, axis=0, keepdims=True)
        c_acc = t_acc * 8 + sub
        cbest = jnp.min(jnp.where(k_acc == m, c_acc, jnp.int32(1 << 30)),
                        axis=0, keepdims=True)
        log_prob = jnp.sum(jnp.where(c_acc == cbest, lp_acc, 0.0),
                           axis=0, keepdims=True)
        out_ref[n, :] = (log_prob + ent)[0, :]


@jax.jit
def kernel(dist, gumbel_u):
    dist_t = jnp.transpose(dist)
    gu_t = jnp.transpose(gumbel_u, (0, 2, 1))
    return pl.pallas_call(
        _sampler_kernel,
        grid=(_NB,),
        in_specs=[
            pl.BlockSpec((_C, _BB), lambda i: (0, i)),
            pl.BlockSpec((_N_SAMPLES, _C, _BB), lambda i: (0, 0, i)),
        ],
        out_specs=pl.BlockSpec((_N_SAMPLES, _BB), lambda i: (0, i)),
        out_shape=jax.ShapeDtypeStruct((_N_SAMPLES, _BATCH), jnp.float32),
    )(dist_t, gu_t)
